# baseline (speedup 1.0000x reference)
import functools

import jax
import jax.numpy as jnp
from jax import lax
from jax.experimental import pallas as pl
from jax.experimental.pallas import tpu as pltpu
from jax.experimental.pallas import tpu_sc as plsc

NC = 2
NS = 16
NW = NC * NS
CH = 128


def _sc_scatter_call(d, nseg, n_pad):
    rpz = n_pad // NS
    zfull = rpz // CH
    zrem = rpz % CH

    mesh = plsc.VectorSubcoreMesh(
        core_axis_name="c", subcore_axis_name="s", num_cores=NC,
        num_subcores=NS)

    @functools.partial(
        pl.kernel,
        mesh=mesh,
        out_type=jax.ShapeDtypeStruct((NC, n_pad, d), jnp.float32),
        scratch_types=[
            pltpu.VMEM((nseg, CH), jnp.int32),
            pltpu.VMEM((nseg, CH), jnp.int32),
            pltpu.VMEM((CH, d), jnp.float32),
            pltpu.VMEM_SHARED((n_pad, d), jnp.float32),
            pltpu.SemaphoreType.DMA,
        ],
    )
    def scatter_kernel(rows_hbm, src_hbm, dst_hbm, out_hbm,
                       src_v, dst_v, rows_v, acc_sh, sem):
        c = lax.axis_index("c")
        s = lax.axis_index("s")

        zero16 = jnp.zeros((16,), jnp.float32)

        def zrow(i, carry):
            for j in range(d // 16):
                rows_v[i, pl.ds(j * 16, 16)] = zero16
            return carry

        lax.fori_loop(0, CH, zrow, 0)
        for k in range(zfull):
            pltpu.sync_copy(rows_v, acc_sh.at[pl.ds(s * rpz + k * CH, CH)])
        if zrem:
            pltpu.sync_copy(
                rows_v.at[pl.ds(0, zrem)],
                acc_sh.at[pl.ds(s * rpz + zfull * CH, zrem)])
        plsc.subcore_barrier()

        def run_seg(slot):
            pltpu.sync_copy(src_hbm.at[s, slot], src_v)
            pltpu.sync_copy(dst_hbm.at[s, slot], dst_v)

            def step(j, carry):
                pltpu.async_copy(rows_hbm.at[src_v.at[j]], rows_v,
                                 sem).wait()
                pltpu.sync_copy(rows_v, acc_sh.at[dst_v.at[j]], add=True)
                return carry

            lax.fori_loop(0, nseg, step, 0)

        @pl.when(c == 1)
        def _():
            run_seg(0)
            run_seg(1)

        @pl.when(c == 0)
        def _():
            run_seg(2)

        plsc.subcore_barrier()

        pltpu.sync_copy(acc_sh.at[pl.ds(s * rpz, rpz)],
                        out_hbm.at[c, pl.ds(s * rpz, rpz)])

    return scatter_kernel


def _combine_matmul_relu_call(p, w, rows_blk):
    _, n, d = p.shape

    def body(p_ref, w_ref, o_ref):
        agg = p_ref[0] + p_ref[1]
        o_ref[...] = jnp.maximum(
            jnp.dot(agg, w_ref[...], preferred_element_type=jnp.float32), 0.0)

    return pl.pallas_call(
        body,
        grid=(n // rows_blk,),
        in_specs=[
            pl.BlockSpec((NC, rows_blk, d), lambda i: (0, i, 0)),
            pl.BlockSpec((d, d), lambda i: (0, 0)),
        ],
        out_specs=pl.BlockSpec((rows_blk, d), lambda i: (i, 0)),
        out_shape=jax.ShapeDtypeStruct((n, d), jnp.float32),
    )(p, w)


def kernel(A, X, W1, W2):
    x = X[0]
    n, d = x.shape
    e = A.shape[1]

    nseg = -(-e // (NS * 3 * CH))
    e_pad = NS * 3 * nseg * CH
    n_pad = -(-(n + 1) // 128) * 128

    trash = n + jnp.arange(e_pad - e, dtype=jnp.int32) % (n_pad - n)
    src = jnp.concatenate(
        [A[0], jnp.zeros((e_pad - e,), jnp.int32)]).reshape(NS, 3, nseg, CH)
    dst = jnp.concatenate([A[1], trash]).reshape(NS, 3, nseg, CH)

    scatter = _sc_scatter_call(d, nseg, n_pad)

    blk = n_pad // 8
    p1 = scatter(x, src, dst)
    h1 = _combine_matmul_relu_call(p1, W1, blk)
    p2 = scatter(h1, src, dst)
    out = _combine_matmul_relu_call(p2, W2, blk)
    return out[None, :n, :]


# speedup vs baseline: 2.3111x; 2.3111x over previous
import functools

import jax
import jax.numpy as jnp
from jax import lax
from jax.experimental import pallas as pl
from jax.experimental.pallas import tpu as pltpu
from jax.experimental.pallas import tpu_sc as plsc

NC = 2
NS = 16
NW = NC * NS
CH = 128


def _sc_scatter_call(d, tchunks, n_pad):
    c0 = (tchunks // NW) // 8 * 8
    rem = tchunks - NW * c0
    nb = rem // 8
    tail = rem - 8 * nb

    rpz = n_pad // NS
    zfull = rpz // CH
    zrem = rpz % CH

    mesh = plsc.VectorSubcoreMesh(
        core_axis_name="c", subcore_axis_name="s", num_cores=NC,
        num_subcores=NS)

    @functools.partial(
        pl.kernel,
        mesh=mesh,
        out_type=jax.ShapeDtypeStruct((NC, n_pad, d), jnp.float32),
        scratch_types=[
            pltpu.VMEM((c0 + 8, CH), jnp.int32),
            pltpu.VMEM((c0 + 8, CH), jnp.int32),
            pltpu.VMEM((CH, d), jnp.float32),
            pltpu.VMEM_SHARED((n_pad, d), jnp.float32),
            pltpu.SemaphoreType.DMA,
        ],
    )
    def scatter_kernel(rows_hbm, a_hbm, out_hbm,
                       src_v, dst_v, rows_v, acc_sh, sem):
        c = lax.axis_index("c")
        s = lax.axis_index("s")
        w = s * NC + c

        zero16 = jnp.zeros((16,), jnp.float32)

        def zrow(i, carry):
            for j in range(d // 16):
                rows_v[i, pl.ds(j * 16, 16)] = zero16
            return carry

        lax.fori_loop(0, CH, zrow, 0)
        for k in range(zfull):
            pltpu.sync_copy(rows_v, acc_sh.at[pl.ds(s * rpz + k * CH, CH)])
        if zrem:
            pltpu.sync_copy(
                rows_v.at[pl.ds(0, zrem)],
                acc_sh.at[pl.ds(s * rpz + zfull * CH, zrem)])
        plsc.subcore_barrier()

        def step(j, carry):
            pltpu.async_copy(rows_hbm.at[src_v.at[j]], rows_v, sem).wait()
            pltpu.sync_copy(rows_v, acc_sh.at[dst_v.at[j]], add=True)
            return carry

        big = w < nb
        base = jnp.where(big, w * (c0 + 8), nb * 8 + w * c0)
        nch = jnp.where(big, c0 + 8, c0)
        pltpu.sync_copy(a_hbm.at[0, pl.ds(base, c0)], src_v.at[pl.ds(0, c0)])
        pltpu.sync_copy(a_hbm.at[1, pl.ds(base, c0)], dst_v.at[pl.ds(0, c0)])

        @pl.when(big)
        def _():
            pltpu.sync_copy(a_hbm.at[0, pl.ds(base + c0, 8)],
                            src_v.at[pl.ds(c0, 8)])
            pltpu.sync_copy(a_hbm.at[1, pl.ds(base + c0, 8)],
                            dst_v.at[pl.ds(c0, 8)])

        lax.fori_loop(0, nch, step, 0)

        if tail:
            @pl.when(w == 0)
            def _():
                pltpu.sync_copy(a_hbm.at[0, pl.ds(tchunks - tail, tail)],
                                src_v.at[pl.ds(0, tail)])
                pltpu.sync_copy(a_hbm.at[1, pl.ds(tchunks - tail, tail)],
                                dst_v.at[pl.ds(0, tail)])
                lax.fori_loop(0, tail, step, 0)

        plsc.subcore_barrier()

        pltpu.sync_copy(acc_sh.at[pl.ds(s * rpz, rpz)],
                        out_hbm.at[c, pl.ds(s * rpz, rpz)])

    return scatter_kernel


def _combine_matmul_relu_call(p, w, out_rows, rows_blk):
    _, _, d = p.shape

    def body(p_ref, w_ref, o_ref):
        agg = p_ref[0] + p_ref[1]
        o_ref[...] = jnp.maximum(
            jnp.dot(agg, w_ref[...], preferred_element_type=jnp.float32), 0.0)

    return pl.pallas_call(
        body,
        grid=(out_rows // rows_blk,),
        in_specs=[
            pl.BlockSpec((NC, rows_blk, d), lambda i: (0, i, 0)),
            pl.BlockSpec((d, d), lambda i: (0, 0)),
        ],
        out_specs=pl.BlockSpec((rows_blk, d), lambda i: (i, 0)),
        out_shape=jax.ShapeDtypeStruct((out_rows, d), jnp.float32),
    )(p, w)


def kernel(A, X, W1, W2):
    x = X[0]
    n, d = x.shape
    e = A.shape[1]

    tchunks = e // CH
    a_chunks = A[:, :tchunks * CH].reshape(2, tchunks, CH)
    n_pad = -(-(n + 1) // 128) * 128

    scatter = _sc_scatter_call(d, tchunks, n_pad)

    p1 = scatter(x, a_chunks)
    h1 = _combine_matmul_relu_call(p1, W1, n, 1000)
    p2 = scatter(h1, a_chunks)
    out = _combine_matmul_relu_call(p2, W2, n, 1000)
    return out[None, :, :]


# speedup vs baseline: 3.5384x; 1.5310x over previous
import functools

import jax
import jax.numpy as jnp
from jax import lax
from jax.experimental import pallas as pl
from jax.experimental.pallas import tpu as pltpu
from jax.experimental.pallas import tpu_sc as plsc

NC = 2
NS = 16
NW = NC * NS
CH = 128


def _sc_scatter_call(d, tchunks, n_pad):
    c0 = (tchunks // NW) // 8 * 8
    rem = tchunks - NW * c0
    nb = rem // 8
    tail = rem - 8 * nb

    rpz = n_pad // NS
    zfull = rpz // CH
    zrem = rpz % CH

    mesh = plsc.VectorSubcoreMesh(
        core_axis_name="c", subcore_axis_name="s", num_cores=NC,
        num_subcores=NS)

    SG = 4

    @functools.partial(
        pl.kernel,
        mesh=mesh,
        out_type=jax.ShapeDtypeStruct((NC, n_pad, d), jnp.float32),
        scratch_types=[
            pltpu.VMEM((SG, CH), jnp.int32),
            pltpu.VMEM((SG, CH), jnp.int32),
            pltpu.VMEM((SG, CH), jnp.int32),
            pltpu.VMEM((SG, CH), jnp.int32),
            pltpu.VMEM((CH, d), jnp.float32),
            pltpu.VMEM((CH, d), jnp.float32),
            pltpu.VMEM_SHARED((n_pad, d), jnp.float32),
            pltpu.SemaphoreType.DMA,
            pltpu.SemaphoreType.DMA,
            pltpu.SemaphoreType.DMA,
            pltpu.SemaphoreType.DMA,
        ],
    )
    def scatter_kernel(rows_hbm, a_hbm, out_hbm,
                       sA, dA, sB, dB, rbuf0, rbuf1, acc_sh,
                       gsem0, gsem1, isemA, isemB):
        c = lax.axis_index("c")
        s = lax.axis_index("s")
        w = s * NC + c

        zero16 = jnp.zeros((16,), jnp.float32)

        def zrow(i, carry):
            for j in range(d // 16):
                rbuf0[i, pl.ds(j * 16, 16)] = zero16
            return carry

        lax.fori_loop(0, CH, zrow, 0)
        for k in range(zfull):
            pltpu.sync_copy(rbuf0, acc_sh.at[pl.ds(s * rpz + k * CH, CH)])
        if zrem:
            pltpu.sync_copy(
                rbuf0.at[pl.ds(0, zrem)],
                acc_sh.at[pl.ds(s * rpz + zfull * CH, zrem)])
        plsc.subcore_barrier()

        big = w < nb
        base = jnp.where(big, w * (c0 + 8), nb * 8 + w * c0)
        nch = jnp.where(big, c0 + 8, c0)
        rbufs = (rbuf0, rbuf1)
        gsems = (gsem0, gsem1)

        def stage(seg_off, sv, dv, isem):
            pltpu.async_copy(a_hbm.at[0, pl.ds(seg_off, SG)], sv, isem)
            pltpu.async_copy(a_hbm.at[1, pl.ds(seg_off, SG)], dv, isem)

        def wait_stage(sv, dv, isem):
            pltpu.make_async_copy(a_hbm.at[0, pl.ds(0, SG)], sv, isem).wait()
            pltpu.make_async_copy(a_hbm.at[1, pl.ds(0, SG)], dv, isem).wait()

        def gath(sv, k, p):
            pltpu.async_copy(rows_hbm.at[sv.at[k]], rbufs[p], gsems[p])

        def wait_gath(p):
            pltpu.make_async_copy(
                rows_hbm.at[sA.at[0]], rbufs[p], gsems[p]).wait()

        def run_seg(sv, dv, nsv, nsv_wait, stage_done):
            for k in range(SG):
                p = k % 2
                if k == SG - 1:
                    nsv_wait()
                    gath(nsv, 0, 1 - p)
                else:
                    gath(sv, k + 1, 1 - p)
                wait_gath(p)
                pltpu.sync_copy(rbufs[p], acc_sh.at[dv.at[k]], add=True)
            stage_done()

        pltpu.sync_copy(a_hbm.at[0, pl.ds(base, SG)], sA)
        pltpu.sync_copy(a_hbm.at[1, pl.ds(base, SG)], dA)
        stage(base + SG, sB, dB, isemB)
        gath(sA, 0, 0)

        def body(i, carry):
            off = base + 2 * i * SG
            off2 = jnp.minimum(off + 2 * SG, tchunks - SG)
            off3 = jnp.minimum(off + 3 * SG, tchunks - SG)
            run_seg(sA, dA, sB,
                    lambda: wait_stage(sB, dB, isemB),
                    lambda: stage(off2, sA, dA, isemA))
            run_seg(sB, dB, sA,
                    lambda: wait_stage(sA, dA, isemA),
                    lambda: stage(off3, sB, dB, isemB))
            return carry

        lax.fori_loop(0, nch // (2 * SG), body, 0)

        wait_gath(0)
        wait_stage(sB, dB, isemB)

        if tail:
            @pl.when(w == 0)
            def _():
                pltpu.sync_copy(a_hbm.at[0, pl.ds(tchunks - tail, tail)],
                                sA.at[pl.ds(0, tail)])
                pltpu.sync_copy(a_hbm.at[1, pl.ds(tchunks - tail, tail)],
                                dA.at[pl.ds(0, tail)])

                def tstep(j, carry):
                    pltpu.async_copy(rows_hbm.at[sA.at[j]], rbuf0,
                                     gsem0).wait()
                    pltpu.sync_copy(rbuf0, acc_sh.at[dA.at[j]], add=True)
                    return carry

                lax.fori_loop(0, tail, tstep, 0)

        plsc.subcore_barrier()

        pltpu.sync_copy(acc_sh.at[pl.ds(s * rpz, rpz)],
                        out_hbm.at[c, pl.ds(s * rpz, rpz)])

    return scatter_kernel


def _combine_matmul_relu_call(p, w, out_rows, rows_blk):
    _, _, d = p.shape

    def body(p_ref, w_ref, o_ref):
        agg = p_ref[0] + p_ref[1]
        o_ref[...] = jnp.maximum(
            jnp.dot(agg, w_ref[...], preferred_element_type=jnp.float32), 0.0)

    return pl.pallas_call(
        body,
        grid=(out_rows // rows_blk,),
        in_specs=[
            pl.BlockSpec((NC, rows_blk, d), lambda i: (0, i, 0)),
            pl.BlockSpec((d, d), lambda i: (0, 0)),
        ],
        out_specs=pl.BlockSpec((rows_blk, d), lambda i: (i, 0)),
        out_shape=jax.ShapeDtypeStruct((out_rows, d), jnp.float32),
    )(p, w)


def kernel(A, X, W1, W2):
    x = X[0]
    n, d = x.shape
    e = A.shape[1]

    tchunks = e // CH
    a_chunks = A[:, :tchunks * CH].reshape(2, tchunks, CH)
    n_pad = -(-(n + 1) // 128) * 128

    scatter = _sc_scatter_call(d, tchunks, n_pad)

    p1 = scatter(x, a_chunks)
    h1 = _combine_matmul_relu_call(p1, W1, n, 1000)
    p2 = scatter(h1, a_chunks)
    out = _combine_matmul_relu_call(p2, W2, n, 1000)
    return out[None, :, :]


# speedup vs baseline: 3.7477x; 1.0592x over previous
import functools

import jax
import jax.numpy as jnp
from jax import lax
from jax.experimental import pallas as pl
from jax.experimental.pallas import tpu as pltpu
from jax.experimental.pallas import tpu_sc as plsc

NC = 2
NS = 16
NW = NC * NS
CH = 128


def _sc_scatter_call(d, tchunks, n_pad):
    c0 = (tchunks // NW) // 8 * 8
    rem = tchunks - NW * c0
    nb = rem // 8
    tail = rem - 8 * nb

    rpz = n_pad // NS
    zfull = rpz // CH
    zrem = rpz % CH

    mesh = plsc.VectorSubcoreMesh(
        core_axis_name="c", subcore_axis_name="s", num_cores=NC,
        num_subcores=NS)

    SG = 4

    @functools.partial(
        pl.kernel,
        mesh=mesh,
        out_type=jax.ShapeDtypeStruct((NC, n_pad, d), jnp.float32),
        scratch_types=[
            pltpu.VMEM((SG, CH), jnp.int32),
            pltpu.VMEM((SG, CH), jnp.int32),
            pltpu.VMEM((SG, CH), jnp.int32),
            pltpu.VMEM((SG, CH), jnp.int32),
            pltpu.VMEM((CH, d), jnp.float32),
            pltpu.VMEM((CH, d), jnp.float32),
            pltpu.VMEM_SHARED((n_pad, d), jnp.float32),
            pltpu.SemaphoreType.DMA,
            pltpu.SemaphoreType.DMA,
            pltpu.SemaphoreType.DMA,
            pltpu.SemaphoreType.DMA,
        ],
    )
    def scatter_kernel(rows_hbm, a_hbm, out_hbm,
                       sA, dA, sB, dB, rbuf0, rbuf1, acc_sh,
                       gsem0, gsem1, isemA, isemB):
        c = lax.axis_index("c")
        s = lax.axis_index("s")
        w = s * NC + c

        zero16 = jnp.zeros((16,), jnp.float32)

        def zrow(i, carry):
            for j in range(d // 16):
                rbuf0[i, pl.ds(j * 16, 16)] = zero16
            return carry

        lax.fori_loop(0, CH, zrow, 0)
        for k in range(zfull):
            pltpu.sync_copy(rbuf0, acc_sh.at[pl.ds(s * rpz + k * CH, CH)])
        if zrem:
            pltpu.sync_copy(
                rbuf0.at[pl.ds(0, zrem)],
                acc_sh.at[pl.ds(s * rpz + zfull * CH, zrem)])
        plsc.subcore_barrier()

        big = w < nb
        base = jnp.where(big, w * (c0 + 8), nb * 8 + w * c0)
        nch = jnp.where(big, c0 + 8, c0)
        rbufs = (rbuf0, rbuf1)
        gsems = (gsem0, gsem1)

        def stage(seg_off, sv, dv, isem):
            pltpu.async_copy(a_hbm.at[0, pl.ds(seg_off, SG)], sv, isem)
            pltpu.async_copy(a_hbm.at[1, pl.ds(seg_off, SG)], dv, isem)

        def wait_stage(sv, dv, isem):
            pltpu.make_async_copy(a_hbm.at[0, pl.ds(0, SG)], sv, isem).wait()
            pltpu.make_async_copy(a_hbm.at[1, pl.ds(0, SG)], dv, isem).wait()

        def gath(sv, k, p):
            pltpu.async_copy(rows_hbm.at[sv.at[k]], rbufs[p], gsems[p])

        def wait_gath(p):
            pltpu.make_async_copy(
                rows_hbm.at[sA.at[0]], rbufs[p], gsems[p]).wait()

        def run_seg(sv, dv, nsv, nsv_wait, stage_done):
            for k in range(SG):
                p = k % 2
                if k == SG - 1:
                    nsv_wait()
                    gath(nsv, 0, 1 - p)
                else:
                    gath(sv, k + 1, 1 - p)
                wait_gath(p)
                pltpu.sync_copy(rbufs[p], acc_sh.at[dv.at[k]], add=True)
            stage_done()

        pltpu.sync_copy(a_hbm.at[0, pl.ds(base, SG)], sA)
        pltpu.sync_copy(a_hbm.at[1, pl.ds(base, SG)], dA)
        stage(base + SG, sB, dB, isemB)
        gath(sA, 0, 0)

        def body(i, carry):
            off = base + 2 * i * SG
            off2 = jnp.minimum(off + 2 * SG, tchunks - SG)
            off3 = jnp.minimum(off + 3 * SG, tchunks - SG)
            run_seg(sA, dA, sB,
                    lambda: wait_stage(sB, dB, isemB),
                    lambda: stage(off2, sA, dA, isemA))
            run_seg(sB, dB, sA,
                    lambda: wait_stage(sA, dA, isemA),
                    lambda: stage(off3, sB, dB, isemB))
            return carry

        lax.fori_loop(0, nch // (2 * SG), body, 0)

        wait_gath(0)
        wait_stage(sB, dB, isemB)

        if tail:
            tw = w - nb

            @pl.when((w >= nb) & (w < nb + tail))
            def _():
                toff = tchunks - tail + tw
                pltpu.sync_copy(a_hbm.at[0, pl.ds(toff, 1)],
                                sA.at[pl.ds(0, 1)])
                pltpu.sync_copy(a_hbm.at[1, pl.ds(toff, 1)],
                                dA.at[pl.ds(0, 1)])
                pltpu.async_copy(rows_hbm.at[sA.at[0]], rbuf0,
                                 gsem0).wait()
                pltpu.sync_copy(rbuf0, acc_sh.at[dA.at[0]], add=True)

        plsc.subcore_barrier()

        pltpu.sync_copy(acc_sh.at[pl.ds(s * rpz, rpz)],
                        out_hbm.at[c, pl.ds(s * rpz, rpz)])

    return scatter_kernel


def _combine_matmul_relu_call(p, w, out_rows, rows_blk):
    _, _, d = p.shape

    def body(p_ref, w_ref, o_ref):
        agg = p_ref[0] + p_ref[1]
        o_ref[...] = jnp.maximum(
            jnp.dot(agg, w_ref[...], preferred_element_type=jnp.float32), 0.0)

    return pl.pallas_call(
        body,
        grid=(out_rows // rows_blk,),
        in_specs=[
            pl.BlockSpec((NC, rows_blk, d), lambda i: (0, i, 0)),
            pl.BlockSpec((d, d), lambda i: (0, 0)),
        ],
        out_specs=pl.BlockSpec((rows_blk, d), lambda i: (i, 0)),
        out_shape=jax.ShapeDtypeStruct((out_rows, d), jnp.float32),
    )(p, w)


def kernel(A, X, W1, W2):
    x = X[0]
    n, d = x.shape
    e = A.shape[1]

    tchunks = e // CH
    a_chunks = A[:, :tchunks * CH].reshape(2, tchunks, CH)
    n_pad = -(-(n + 1) // 128) * 128

    scatter = _sc_scatter_call(d, tchunks, n_pad)

    p1 = scatter(x, a_chunks)
    h1 = _combine_matmul_relu_call(p1, W1, n, 1000)
    p2 = scatter(h1, a_chunks)
    out = _combine_matmul_relu_call(p2, W2, n, 1000)
    return out[None, :, :]


# speedup vs baseline: 3.8260x; 1.0209x over previous
import functools

import jax
import jax.numpy as jnp
from jax import lax
from jax.experimental import pallas as pl
from jax.experimental.pallas import tpu as pltpu
from jax.experimental.pallas import tpu_sc as plsc

NC = 2
NS = 16
NW = NC * NS
CH = 128


def _sc_scatter_call(d, tchunks, n_pad):
    c0 = (tchunks // NW) // 8 * 8
    rem = tchunks - NW * c0
    nb = rem // 8
    tail = rem - 8 * nb

    rpz = n_pad // NS
    zfull = rpz // CH
    zrem = rpz % CH

    mesh = plsc.VectorSubcoreMesh(
        core_axis_name="c", subcore_axis_name="s", num_cores=NC,
        num_subcores=NS)

    SG = 4

    @functools.partial(
        pl.kernel,
        mesh=mesh,
        out_type=jax.ShapeDtypeStruct((NC, n_pad, d), jnp.float32),
        scratch_types=[
            pltpu.VMEM((SG, CH), jnp.int32),
            pltpu.VMEM((SG, CH), jnp.int32),
            pltpu.VMEM((SG, CH), jnp.int32),
            pltpu.VMEM((SG, CH), jnp.int32),
            pltpu.VMEM((CH, d), jnp.float32),
            pltpu.VMEM((CH, d), jnp.float32),
            pltpu.VMEM_SHARED((n_pad, d), jnp.float32),
            pltpu.SemaphoreType.DMA,
            pltpu.SemaphoreType.DMA,
            pltpu.SemaphoreType.DMA,
            pltpu.SemaphoreType.DMA,
        ],
    )
    def scatter_kernel(rows_hbm, a_hbm, out_hbm,
                       sA, dA, sB, dB, rbuf0, rbuf1, acc_sh,
                       gsem0, gsem1, isemA, isemB):
        c = lax.axis_index("c")
        s = lax.axis_index("s")
        w = s * NC + c

        zero16 = jnp.zeros((16,), jnp.float32)

        def zrow(i, carry):
            for j in range(d // 16):
                rbuf0[i, pl.ds(j * 16, 16)] = zero16
            return carry

        lax.fori_loop(0, CH, zrow, 0)
        for k in range(zfull):
            pltpu.sync_copy(rbuf0, acc_sh.at[pl.ds(s * rpz + k * CH, CH)])
        if zrem:
            pltpu.sync_copy(
                rbuf0.at[pl.ds(0, zrem)],
                acc_sh.at[pl.ds(s * rpz + zfull * CH, zrem)])
        plsc.subcore_barrier()

        big = w < nb
        base = jnp.where(big, w * (c0 + 8), nb * 8 + w * c0)
        nch = jnp.where(big, c0 + 8, c0)
        rbufs = (rbuf0, rbuf1)
        gsems = (gsem0, gsem1)

        def stage(seg_off, sv, dv, isem):
            pltpu.async_copy(a_hbm.at[0, pl.ds(seg_off, SG)], sv, isem)
            pltpu.async_copy(a_hbm.at[1, pl.ds(seg_off, SG)], dv, isem)

        def wait_stage(sv, dv, isem):
            pltpu.make_async_copy(a_hbm.at[0, pl.ds(0, SG)], sv, isem).wait()
            pltpu.make_async_copy(a_hbm.at[1, pl.ds(0, SG)], dv, isem).wait()

        def gath(sv, k, p):
            pltpu.async_copy(rows_hbm.at[sv.at[k]], rbufs[p], gsems[p])

        def wait_gath(p):
            pltpu.make_async_copy(
                rows_hbm.at[sA.at[0]], rbufs[p], gsems[p]).wait()

        def run_seg(sv, dv, nsv, nsv_wait, stage_done):
            for k in range(SG):
                p = k % 2
                if k == SG - 1:
                    nsv_wait()
                    gath(nsv, 0, 1 - p)
                else:
                    gath(sv, k + 1, 1 - p)
                wait_gath(p)
                pltpu.sync_copy(rbufs[p], acc_sh.at[dv.at[k]], add=True)
            stage_done()

        pltpu.sync_copy(a_hbm.at[0, pl.ds(base, SG)], sA)
        pltpu.sync_copy(a_hbm.at[1, pl.ds(base, SG)], dA)
        stage(base + SG, sB, dB, isemB)
        gath(sA, 0, 0)

        def body(i, carry):
            off = base + 2 * i * SG
            off2 = jnp.minimum(off + 2 * SG, tchunks - SG)
            off3 = jnp.minimum(off + 3 * SG, tchunks - SG)
            run_seg(sA, dA, sB,
                    lambda: wait_stage(sB, dB, isemB),
                    lambda: stage(off2, sA, dA, isemA))
            run_seg(sB, dB, sA,
                    lambda: wait_stage(sA, dA, isemA),
                    lambda: stage(off3, sB, dB, isemB))
            return carry

        lax.fori_loop(0, nch // (2 * SG), body, 0)

        wait_gath(0)
        wait_stage(sB, dB, isemB)

        if tail:
            tw = w - nb

            @pl.when((w >= nb) & (w < nb + tail))
            def _():
                toff = tchunks - tail + tw
                pltpu.sync_copy(a_hbm.at[0, pl.ds(toff, 1)],
                                sA.at[pl.ds(0, 1)])
                pltpu.sync_copy(a_hbm.at[1, pl.ds(toff, 1)],
                                dA.at[pl.ds(0, 1)])
                pltpu.async_copy(rows_hbm.at[sA.at[0]], rbuf0,
                                 gsem0).wait()
                pltpu.sync_copy(rbuf0, acc_sh.at[dA.at[0]], add=True)

        plsc.subcore_barrier()

        pltpu.sync_copy(acc_sh.at[pl.ds(s * rpz, rpz)],
                        out_hbm.at[c, pl.ds(s * rpz, rpz)])

    return scatter_kernel


def _combine_matmul_relu_call(p, w, out_rows, rows_blk):
    _, _, d = p.shape

    def body(p_ref, w_ref, o_ref):
        agg = p_ref[0] + p_ref[1]
        o_ref[...] = jnp.maximum(
            jnp.dot(agg, w_ref[...], preferred_element_type=jnp.float32), 0.0)

    return pl.pallas_call(
        body,
        grid=(out_rows // rows_blk,),
        in_specs=[
            pl.BlockSpec((NC, rows_blk, d), lambda i: (0, i, 0)),
            pl.BlockSpec((d, d), lambda i: (0, 0)),
        ],
        out_specs=pl.BlockSpec((rows_blk, d), lambda i: (i, 0)),
        out_shape=jax.ShapeDtypeStruct((out_rows, d), jnp.float32),
    )(p, w)


def kernel(A, X, W1, W2):
    x = X[0]
    n, d = x.shape
    e = A.shape[1]

    tchunks = e // CH
    a_chunks = A[:, :tchunks * CH].reshape(2, tchunks, CH)
    n_pad = -(-(n + 1) // 128) * 128

    scatter = _sc_scatter_call(d, tchunks, n_pad)

    p1 = scatter(x, a_chunks)
    h1 = _combine_matmul_relu_call(p1, W1, n, 2000)
    p2 = scatter(h1, a_chunks)
    out = _combine_matmul_relu_call(p2, W2, n, 2000)
    return out[None, :, :]


# speedup vs baseline: 3.8682x; 1.0110x over previous
import functools

import jax
import jax.numpy as jnp
from jax import lax
from jax.experimental import pallas as pl
from jax.experimental.pallas import tpu as pltpu
from jax.experimental.pallas import tpu_sc as plsc

NC = 2
NS = 16
NW = NC * NS
CH = 128


def _sc_scatter_call(d, tchunks, n_pad):
    c0 = (tchunks // NW) // 8 * 8
    rem = tchunks - NW * c0
    nb = rem // 8
    tail = rem - 8 * nb

    rpz = n_pad // NS
    zfull = rpz // CH
    zrem = rpz % CH

    mesh = plsc.VectorSubcoreMesh(
        core_axis_name="c", subcore_axis_name="s", num_cores=NC,
        num_subcores=NS)

    SG = 4

    @functools.partial(
        pl.kernel,
        mesh=mesh,
        out_type=jax.ShapeDtypeStruct((NC, n_pad, d), jnp.float32),
        scratch_types=[
            pltpu.VMEM((SG, CH), jnp.int32),
            pltpu.VMEM((SG, CH), jnp.int32),
            pltpu.VMEM((SG, CH), jnp.int32),
            pltpu.VMEM((SG, CH), jnp.int32),
            pltpu.VMEM((CH, d), jnp.float32),
            pltpu.VMEM((CH, d), jnp.float32),
            pltpu.VMEM_SHARED((n_pad, d), jnp.float32),
            pltpu.SemaphoreType.DMA,
            pltpu.SemaphoreType.DMA,
            pltpu.SemaphoreType.DMA,
            pltpu.SemaphoreType.DMA,
        ],
    )
    def scatter_kernel(rows_hbm, a_hbm, out_hbm,
                       sA, dA, sB, dB, rbuf0, rbuf1, acc_sh,
                       gsem0, gsem1, isemA, isemB):
        c = lax.axis_index("c")
        s = lax.axis_index("s")
        w = s * NC + c

        zero16 = jnp.zeros((16,), jnp.float32)

        def zrow(i, carry):
            for j in range(d // 16):
                rbuf0[i, pl.ds(j * 16, 16)] = zero16
            return carry

        lax.fori_loop(0, CH, zrow, 0)
        for k in range(zfull):
            pltpu.sync_copy(rbuf0, acc_sh.at[pl.ds(s * rpz + k * CH, CH)])
        if zrem:
            pltpu.sync_copy(
                rbuf0.at[pl.ds(0, zrem)],
                acc_sh.at[pl.ds(s * rpz + zfull * CH, zrem)])
        plsc.subcore_barrier()

        big = w < nb
        base = jnp.where(big, w * (c0 + 8), nb * 8 + w * c0)
        nch = jnp.where(big, c0 + 8, c0)
        rbufs = (rbuf0, rbuf1)
        gsems = (gsem0, gsem1)

        def stage(seg_off, sv, dv, isem):
            pltpu.async_copy(a_hbm.at[0, pl.ds(seg_off, SG)], sv, isem)
            pltpu.async_copy(a_hbm.at[1, pl.ds(seg_off, SG)], dv, isem)

        def wait_stage(sv, dv, isem):
            pltpu.make_async_copy(a_hbm.at[0, pl.ds(0, SG)], sv, isem).wait()
            pltpu.make_async_copy(a_hbm.at[1, pl.ds(0, SG)], dv, isem).wait()

        def gath(sv, k, p):
            pltpu.async_copy(rows_hbm.at[sv.at[k]], rbufs[p], gsems[p])

        def wait_gath(p):
            pltpu.make_async_copy(
                rows_hbm.at[sA.at[0]], rbufs[p], gsems[p]).wait()

        def run_seg(sv, dv, nsv, nsv_wait, stage_done):
            for k in range(SG):
                p = k % 2
                if k == SG - 1:
                    nsv_wait()
                    gath(nsv, 0, 1 - p)
                else:
                    gath(sv, k + 1, 1 - p)
                wait_gath(p)
                pltpu.sync_copy(rbufs[p], acc_sh.at[dv.at[k]], add=True)
            stage_done()

        pltpu.sync_copy(a_hbm.at[0, pl.ds(base, SG)], sA)
        pltpu.sync_copy(a_hbm.at[1, pl.ds(base, SG)], dA)
        stage(base + SG, sB, dB, isemB)
        gath(sA, 0, 0)

        def body(i, carry):
            off = base + 2 * i * SG
            off2 = jnp.minimum(off + 2 * SG, tchunks - SG)
            off3 = jnp.minimum(off + 3 * SG, tchunks - SG)
            run_seg(sA, dA, sB,
                    lambda: wait_stage(sB, dB, isemB),
                    lambda: stage(off2, sA, dA, isemA))
            run_seg(sB, dB, sA,
                    lambda: wait_stage(sA, dA, isemA),
                    lambda: stage(off3, sB, dB, isemB))
            return carry

        lax.fori_loop(0, nch // (2 * SG), body, 0)

        wait_gath(0)
        wait_stage(sB, dB, isemB)

        if tail:
            tw = w - nb

            @pl.when((w >= nb) & (w < nb + tail))
            def _():
                toff = tchunks - tail + tw
                pltpu.sync_copy(a_hbm.at[0, pl.ds(toff, 1)],
                                sA.at[pl.ds(0, 1)])
                pltpu.sync_copy(a_hbm.at[1, pl.ds(toff, 1)],
                                dA.at[pl.ds(0, 1)])
                pltpu.async_copy(rows_hbm.at[sA.at[0]], rbuf0,
                                 gsem0).wait()
                pltpu.sync_copy(rbuf0, acc_sh.at[dA.at[0]], add=True)

        plsc.subcore_barrier()

        pltpu.sync_copy(acc_sh.at[pl.ds(s * rpz, rpz)],
                        out_hbm.at[c, pl.ds(s * rpz, rpz)])

    return scatter_kernel


def _combine_matmul_relu_call(p, w, out_rows, rows_blk):
    _, _, d = p.shape

    def body(p_ref, w_ref, o_ref):
        agg = p_ref[0] + p_ref[1]
        o_ref[...] = jnp.maximum(
            jnp.dot(agg, w_ref[...], preferred_element_type=jnp.float32), 0.0)

    return pl.pallas_call(
        body,
        grid=(out_rows // rows_blk,),
        in_specs=[
            pl.BlockSpec((NC, rows_blk, d), lambda i: (0, i, 0)),
            pl.BlockSpec((d, d), lambda i: (0, 0)),
        ],
        out_specs=pl.BlockSpec((rows_blk, d), lambda i: (i, 0)),
        out_shape=jax.ShapeDtypeStruct((out_rows, d), jnp.float32),
    )(p, w)


def kernel(A, X, W1, W2):
    x = X[0]
    n, d = x.shape
    e = A.shape[1]

    tchunks = e // CH
    a_chunks = A[:, :tchunks * CH].reshape(2, tchunks, CH)
    n_pad = -(-(n + 1) // 128) * 128

    scatter = _sc_scatter_call(d, tchunks, n_pad)

    p1 = scatter(x, a_chunks)
    h1 = _combine_matmul_relu_call(p1, W1, n, 5000)
    p2 = scatter(h1, a_chunks)
    out = _combine_matmul_relu_call(p2, W2, n, 5000)
    return out[None, :, :]
